# baseline (speedup 1.0000x reference)
import math

import jax
import jax.numpy as jnp
from jax import lax
from jax.experimental import pallas as pl
from jax.experimental.pallas import tpu as pltpu
from jax.experimental.pallas import tpu_sc as plsc

VOCAB = 1000000
D = 64
BATCH = 4096
HIST = 200
SCALE = math.sqrt(D)

NC = 2
NS = 16
NW = NC * NS
BW = BATCH // NW
NBUF = 4
LANES = 16
BT = BW // LANES


def _emb_body(xt_hbm, tab_hbm, out_hbm, idx_v, idx2_v, g_buf, t_buf, *sems):
    gsem = sems[:NBUF]
    osem = sems[NBUF:]
    wid = lax.axis_index("s") * NC + lax.axis_index("c")
    b0 = wid * BW

    pltpu.sync_copy(xt_hbm.at[:, pl.ds(b0, BW)], idx_v)

    lane = lax.iota(jnp.int32, LANES)
    rv = [lane + bt * LANES for bt in range(BT)]

    def stage_idx2(h, s):
        for g in range(BT):
            v = idx_v[h, pl.ds(g * LANES, LANES)]
            idx2_v[s, pl.ds(g * LANES, LANES)] = lax.shift_right_logical(v, 1)

    def g_copy(s):
        return pltpu.make_async_copy(
            tab_hbm.at[idx2_v.at[s]], g_buf.at[s], gsem[s])

    def o_copy(h, s):
        return pltpu.make_async_copy(
            t_buf.at[s], out_hbm.at[h, :, pl.ds(b0, BW)], osem[s])

    for s in range(NBUF - 1):
        stage_idx2(s, s)
        g_copy(s).start()

    def outer(it, carry):
        ci = it * NBUF
        for s in range(NBUF):
            h = ci + s
            g_copy(s).wait()

            @pl.when(it > 0)
            def _():
                o_copy(h - NBUF, s).wait()

            cb2 = [
                lax.bitwise_and(idx_v[h, pl.ds(bt * LANES, LANES)], 1) * D
                for bt in range(BT)
            ]

            def d_step(d):
                dg = lax.bitwise_and(lane + d, LANES - 1)
                cbd = [cb2[bt] + dg for bt in range(BT)]
                for f0 in range(D // LANES):
                    frow = dg + (f0 * LANES)
                    for bt in range(BT):
                        vals = plsc.load_gather(
                            g_buf.at[s], [rv[bt], cbd[bt] + (f0 * LANES)])
                        plsc.store_scatter(
                            t_buf.at[s], [frow, rv[bt]], vals * SCALE)

            plsc.parallel_loop(0, LANES, 1)(d_step)

            o_copy(h, s).start()

            ng = h + NBUF - 1
            @pl.when(ng < HIST)
            def _():
                ns = (s + NBUF - 1) % NBUF
                stage_idx2(ng, ns)
                g_copy(ns).start()
        return carry

    lax.fori_loop(0, HIST // NBUF, outer, 0)

    for s in range(NBUF):
        o_copy(HIST - NBUF + s, s).wait()


@jax.jit
def _emb(x_t, table2):
    mesh = plsc.VectorSubcoreMesh(core_axis_name="c", subcore_axis_name="s")
    return pl.kernel(
        _emb_body,
        out_type=jax.ShapeDtypeStruct((HIST, D, BATCH), jnp.float32),
        mesh=mesh,
        scratch_types=[
            pltpu.VMEM((HIST, BW), jnp.int32),
            pltpu.VMEM((NBUF, BW), jnp.int32),
            pltpu.VMEM((NBUF, BW, 2 * D), jnp.float32),
            pltpu.VMEM((NBUF, D, BW), jnp.float32),
        ] + [pltpu.SemaphoreType.DMA] * (2 * NBUF),
        compiler_params=pltpu.CompilerParams(
            use_tc_tiling_on_sc=True, needs_layout_passes=False,
            disable_bounds_checks=True),
    )(x_t, table2)


def kernel(x, lut):
    table2 = lut.reshape(VOCAB // 2, 2 * D)
    out_t = _emb(x.T, table2)
    return jnp.transpose(out_t, (2, 0, 1))


# speedup vs baseline: 1.0972x; 1.0972x over previous
import math

import jax
import jax.numpy as jnp
from jax import lax
from jax.experimental import pallas as pl
from jax.experimental.pallas import tpu as pltpu
from jax.experimental.pallas import tpu_sc as plsc

VOCAB = 1000000
D = 64
BATCH = 4096
HIST = 200
SCALE = math.sqrt(D)

NC = 2
NS = 16
NW = NC * NS
BW = BATCH // NW
NBUF = 4
LANES = 16
BT = BW // LANES


def _emb_body(xt_hbm, tab_hbm, out_hbm, idx_v, idx2_v, g_buf, t_buf, *sems):
    gsem = sems[:NBUF]
    osem = sems[NBUF:]
    wid = lax.axis_index("s") * NC + lax.axis_index("c")
    b0 = wid * BW

    pltpu.sync_copy(xt_hbm.at[:, pl.ds(b0, BW)], idx_v)

    lane = lax.iota(jnp.int32, LANES)
    rv = [lane + bt * LANES for bt in range(BT)]

    def stage_idx2(h, s):
        for g in range(BT):
            v = idx_v[h, pl.ds(g * LANES, LANES)]
            idx2_v[s, pl.ds(g * LANES, LANES)] = lax.shift_right_logical(v, 1)

    def g_copy(s):
        return pltpu.make_async_copy(
            tab_hbm.at[idx2_v.at[s]], g_buf.at[s], gsem[s])

    def o_copy(h, s):
        return pltpu.make_async_copy(
            t_buf.at[s], out_hbm.at[h, :, pl.ds(b0, BW)], osem[s])

    for s in range(NBUF - 1):
        stage_idx2(s, s)
        g_copy(s).start()

    def outer(it, carry):
        ci = it * NBUF
        for s in range(NBUF):
            h = ci + s
            g_copy(s).wait()

            @pl.when(it > 0)
            def _():
                o_copy(h - NBUF, s).wait()

            cb2 = [
                lax.bitwise_and(idx_v[h, pl.ds(bt * LANES, LANES)], 1) * D
                for bt in range(BT)
            ]

            def d_step(d):
                dg = lax.bitwise_and(lane + d, LANES - 1)
                cbd = [cb2[bt] + dg for bt in range(BT)]
                for f0 in range(D // LANES):
                    frow = dg + (f0 * LANES)
                    for bt in range(BT):
                        vals = plsc.load_gather(
                            g_buf.at[s], [rv[bt], cbd[bt] + (f0 * LANES)])
                        plsc.store_scatter(
                            t_buf.at[s], [frow, rv[bt]], vals * SCALE)

            plsc.parallel_loop(0, LANES, 1, unroll=2)(d_step)

            o_copy(h, s).start()

            ng = h + NBUF - 1
            @pl.when(ng < HIST)
            def _():
                ns = (s + NBUF - 1) % NBUF
                stage_idx2(ng, ns)
                g_copy(ns).start()
        return carry

    lax.fori_loop(0, HIST // NBUF, outer, 0)

    for s in range(NBUF):
        o_copy(HIST - NBUF + s, s).wait()


@jax.jit
def _emb(x_t, table2):
    mesh = plsc.VectorSubcoreMesh(core_axis_name="c", subcore_axis_name="s")
    return pl.kernel(
        _emb_body,
        out_type=jax.ShapeDtypeStruct((HIST, D, BATCH), jnp.float32),
        mesh=mesh,
        scratch_types=[
            pltpu.VMEM((HIST, BW), jnp.int32),
            pltpu.VMEM((NBUF, BW), jnp.int32),
            pltpu.VMEM((NBUF, BW, 2 * D), jnp.float32),
            pltpu.VMEM((NBUF, D, BW), jnp.float32),
        ] + [pltpu.SemaphoreType.DMA] * (2 * NBUF),
        compiler_params=pltpu.CompilerParams(
            use_tc_tiling_on_sc=True, needs_layout_passes=False,
            disable_bounds_checks=True),
    )(x_t, table2)


def kernel(x, lut):
    table2 = lut.reshape(VOCAB // 2, 2 * D)
    out_t = _emb(x.T, table2)
    return jnp.transpose(out_t, (2, 0, 1))


# speedup vs baseline: 1.1072x; 1.0091x over previous
import math

import jax
import jax.numpy as jnp
from jax import lax
from jax.experimental import pallas as pl
from jax.experimental.pallas import tpu as pltpu
from jax.experimental.pallas import tpu_sc as plsc

VOCAB = 1000000
D = 64
BATCH = 4096
HIST = 200
SCALE = math.sqrt(D)

NC = 2
NS = 16
NW = NC * NS
BW = BATCH // NW
NBUF = 4
LANES = 16
BT = BW // LANES


def _emb_body(xt_hbm, tab_hbm, out_hbm, idx_v, idx2_v, g_buf, t_buf, *sems):
    gsem = sems[:NBUF]
    osem = sems[NBUF:]
    wid = lax.axis_index("s") * NC + lax.axis_index("c")
    b0 = wid * BW

    pltpu.sync_copy(xt_hbm.at[:, pl.ds(b0, BW)], idx_v)

    lane = lax.iota(jnp.int32, LANES)
    rv = [lane + bt * LANES for bt in range(BT)]

    def stage_idx2(h, s):
        for g in range(BT):
            v = idx_v[h, pl.ds(g * LANES, LANES)]
            idx2_v[s, pl.ds(g * LANES, LANES)] = lax.shift_right_logical(v, 1)

    def g_copy(s):
        return pltpu.make_async_copy(
            tab_hbm.at[idx2_v.at[s]], g_buf.at[s], gsem[s])

    def o_copy(h, s):
        return pltpu.make_async_copy(
            t_buf.at[s], out_hbm.at[h, :, pl.ds(b0, BW)], osem[s])

    for s in range(NBUF - 1):
        stage_idx2(s, s)
        g_copy(s).start()

    def outer(it, carry):
        ci = it * NBUF
        for s in range(NBUF):
            h = ci + s
            g_copy(s).wait()

            @pl.when(it > 0)
            def _():
                o_copy(h - NBUF, s).wait()

            cb2 = [
                lax.bitwise_and(idx_v[h, pl.ds(bt * LANES, LANES)], 1) * D
                for bt in range(BT)
            ]

            def d_step(d):
                dg = lax.bitwise_and(lane + d, LANES - 1)
                cbd = [cb2[bt] + dg for bt in range(BT)]
                for f0 in range(D // LANES):
                    frow = dg + (f0 * LANES)
                    for bt in range(BT):
                        vals = plsc.load_gather(
                            g_buf.at[s], [rv[bt], cbd[bt] + (f0 * LANES)])
                        plsc.store_scatter(
                            t_buf.at[s], [frow, rv[bt]], vals * SCALE)

            plsc.parallel_loop(0, LANES, 1, unroll=4)(d_step)

            o_copy(h, s).start()

            ng = h + NBUF - 1
            @pl.when(ng < HIST)
            def _():
                ns = (s + NBUF - 1) % NBUF
                stage_idx2(ng, ns)
                g_copy(ns).start()
        return carry

    lax.fori_loop(0, HIST // NBUF, outer, 0)

    for s in range(NBUF):
        o_copy(HIST - NBUF + s, s).wait()


@jax.jit
def _emb(x_t, table2):
    mesh = plsc.VectorSubcoreMesh(core_axis_name="c", subcore_axis_name="s")
    return pl.kernel(
        _emb_body,
        out_type=jax.ShapeDtypeStruct((HIST, D, BATCH), jnp.float32),
        mesh=mesh,
        scratch_types=[
            pltpu.VMEM((HIST, BW), jnp.int32),
            pltpu.VMEM((NBUF, BW), jnp.int32),
            pltpu.VMEM((NBUF, BW, 2 * D), jnp.float32),
            pltpu.VMEM((NBUF, D, BW), jnp.float32),
        ] + [pltpu.SemaphoreType.DMA] * (2 * NBUF),
        compiler_params=pltpu.CompilerParams(
            use_tc_tiling_on_sc=True, needs_layout_passes=False,
            disable_bounds_checks=True),
    )(x_t, table2)


def kernel(x, lut):
    table2 = lut.reshape(VOCAB // 2, 2 * D)
    out_t = _emb(x.T, table2)
    return jnp.transpose(out_t, (2, 0, 1))


# speedup vs baseline: 1.1974x; 1.0815x over previous
import math

import jax
import jax.numpy as jnp
from jax import lax
from jax.experimental import pallas as pl
from jax.experimental.pallas import tpu as pltpu
from jax.experimental.pallas import tpu_sc as plsc

VOCAB = 1000000
D = 64
BATCH = 4096
HIST = 200
SCALE = math.sqrt(D)

NC = 2
NS = 16
NW = NC * NS
BW = BATCH // NW
NBUF = 4
LANES = 16
BT = BW // LANES


def _emb_body(xt_hbm, tab_hbm, out_hbm, idx_v, idx2_v, g_buf, t_buf, *sems):
    gsem = sems[:NBUF]
    osem = sems[NBUF:]
    wid = lax.axis_index("s") * NC + lax.axis_index("c")
    b0 = wid * BW

    pltpu.sync_copy(xt_hbm.at[:, pl.ds(b0, BW)], idx_v)

    lane = lax.iota(jnp.int32, LANES)
    rv = [lane + bt * LANES for bt in range(BT)]

    def stage_idx2(h, s):
        for g in range(BT):
            v = idx_v[h, pl.ds(g * LANES, LANES)]
            idx2_v[s, pl.ds(g * LANES, LANES)] = lax.shift_right_logical(v, 1)

    def g_copy(s):
        return pltpu.make_async_copy(
            tab_hbm.at[idx2_v.at[s]], g_buf.at[s], gsem[s])

    def o_copy(h, s):
        return pltpu.make_async_copy(
            t_buf.at[s], out_hbm.at[h, :, pl.ds(b0, BW)], osem[s])

    for s in range(NBUF - 1):
        stage_idx2(s, s)
        g_copy(s).start()

    def outer(it, carry):
        ci = it * NBUF
        for s in range(NBUF):
            h = ci + s
            g_copy(s).wait()

            @pl.when(it > 0)
            def _():
                o_copy(h - NBUF, s).wait()

            cb2 = [
                lax.bitwise_and(idx_v[h, pl.ds(bt * LANES, LANES)], 1) * D
                for bt in range(BT)
            ]

            def d_step(d):
                dg = lax.bitwise_and(lane + d, LANES - 1)
                cbd = [cb2[bt] + dg for bt in range(BT)]
                for f0 in range(D // LANES):
                    frow = dg + (f0 * LANES)
                    for bt in range(BT):
                        vals = plsc.load_gather(
                            g_buf.at[s], [rv[bt], cbd[bt] + (f0 * LANES)])
                        plsc.store_scatter(
                            t_buf.at[s], [frow, rv[bt]], vals * SCALE)

            plsc.parallel_loop(0, LANES, 1, unroll=8)(d_step)

            o_copy(h, s).start()

            ng = h + NBUF - 1
            @pl.when(ng < HIST)
            def _():
                ns = (s + NBUF - 1) % NBUF
                stage_idx2(ng, ns)
                g_copy(ns).start()
        return carry

    lax.fori_loop(0, HIST // NBUF, outer, 0)

    for s in range(NBUF):
        o_copy(HIST - NBUF + s, s).wait()


@jax.jit
def _emb(x_t, table2):
    mesh = plsc.VectorSubcoreMesh(core_axis_name="c", subcore_axis_name="s")
    return pl.kernel(
        _emb_body,
        out_type=jax.ShapeDtypeStruct((HIST, D, BATCH), jnp.float32),
        mesh=mesh,
        scratch_types=[
            pltpu.VMEM((HIST, BW), jnp.int32),
            pltpu.VMEM((NBUF, BW), jnp.int32),
            pltpu.VMEM((NBUF, BW, 2 * D), jnp.float32),
            pltpu.VMEM((NBUF, D, BW), jnp.float32),
        ] + [pltpu.SemaphoreType.DMA] * (2 * NBUF),
        compiler_params=pltpu.CompilerParams(
            use_tc_tiling_on_sc=True, needs_layout_passes=False,
            disable_bounds_checks=True),
    )(x_t, table2)


def kernel(x, lut):
    table2 = lut.reshape(VOCAB // 2, 2 * D)
    out_t = _emb(x.T, table2)
    return jnp.transpose(out_t, (2, 0, 1))
